# baseline (speedup 1.0000x reference)
import jax
import jax.numpy as jnp
from jax import lax
from jax.experimental import pallas as pl
from jax.experimental.pallas import tpu as pltpu
from jax.experimental.pallas import tpu_sc as plsc

B = 16384
F = 26
DIM = 32
N = B * F
NUM_WORKERS = 32
PER_W = N // NUM_WORKERS
GSZ = 128
K = 13
STEP = K * GSZ
NSTEPS = PER_W // STEP
LANES = 16


def _emb_body(x_hbm, offrep_hbm, table_hbm, out_hbm,
              off_v, raw_v, idx_v, rows_v, sem):
    cid = lax.axis_index("c")
    sid = lax.axis_index("s")
    wid = sid * 2 + cid
    base = wid * PER_W

    pltpu.sync_copy(offrep_hbm, off_v)

    def step(s, carry):
        s_base = base + s * STEP
        pltpu.sync_copy(x_hbm.at[pl.ds(s_base, STEP)], raw_v)
        for j in range(K):
            for c in range(GSZ // LANES):
                p0 = j * GSZ + c * LANES
                idx_v[j, pl.ds(c * LANES, LANES)] = (
                    raw_v[pl.ds(p0, LANES)] + off_v[pl.ds(p0, LANES)]
                )
        cps = [
            pltpu.async_copy(
                table_hbm.at[idx_v.at[j]],
                rows_v.at[pl.ds(j * GSZ, GSZ)],
                sem,
            )
            for j in range(K)
        ]
        for cp in cps:
            cp.wait()
        pltpu.sync_copy(rows_v, out_hbm.at[pl.ds(s_base, STEP)])
        return carry

    lax.fori_loop(0, NSTEPS, step, 0)


@jax.jit
def _emb_call(x_flat, off_rep, table):
    mesh = plsc.VectorSubcoreMesh(core_axis_name="c", subcore_axis_name="s")
    f = pl.kernel(
        _emb_body,
        out_type=jax.ShapeDtypeStruct((N, DIM), jnp.float32),
        mesh=mesh,
        scratch_types=[
            pltpu.VMEM((STEP,), jnp.int32),
            pltpu.VMEM((STEP,), jnp.int32),
            pltpu.VMEM((K, GSZ), jnp.int32),
            pltpu.VMEM((STEP, DIM), jnp.float32),
            pltpu.SemaphoreType.DMA,
        ],
        compiler_params=pltpu.CompilerParams(use_tc_tiling_on_sc=False),
    )
    return f(x_flat, off_rep, table)


def kernel(x_cat, emb_weight, offsets):
    x_flat = x_cat.astype(jnp.int32).reshape(N)
    off_rep = jnp.tile(offsets.astype(jnp.int32), STEP // F)
    out = _emb_call(x_flat, off_rep, emb_weight)
    return out.reshape(B, F, DIM)
